# baseline (speedup 1.0000x reference)
import functools

import jax
import jax.numpy as jnp
from jax import lax
from jax.experimental import pallas as pl
from jax.experimental.pallas import tpu as pltpu
from jax.experimental.pallas import tpu_sc as plsc

_NT = 16384
_D = 2048
_E = 16
_BT = 1024
_STEPS = _NT // _BT
_NW = 32
_CHUNK = _NT // _NW
_GROUPS = _CHUNK // 16


def _matmul_kernel(xa_ref, xb_ref, w_ref, lt_ref):
    w = w_ref[...]
    lta = jax.lax.dot_general(
        w, xa_ref[...], (((1,), (1,)), ((), ())), preferred_element_type=jnp.float32
    )
    ltb = jax.lax.dot_general(
        w, xb_ref[...], (((1,), (1,)), ((), ())), preferred_element_type=jnp.float32
    )
    lt_ref[...] = jnp.concatenate([lta, ltb], axis=1)


def _sc_routing_kernel(lt_hbm, gate_hbm, idx_hbm, part_hbm,
                       lbuf, gbuf, ibuf, accp, accc):
    wid = lax.axis_index("s") * 2 + lax.axis_index("c")
    base = wid * _CHUNK
    pltpu.sync_copy(lt_hbm.at[:, pl.ds(base, _CHUNK)], lbuf)
    zero = jnp.zeros((16,), jnp.float32)
    for e in range(_E):
        accp[e, :] = zero
        accc[e, :] = zero

    def group(g, carry):
        toks = pl.ds(g * 16, 16)
        ls = [lbuf[e, toks] for e in range(_E)]
        m1 = ls[0]
        for e in range(1, _E):
            m1 = jnp.maximum(m1, ls[e])
        idx1 = jnp.full((16,), 16.0, jnp.float32)
        for e in range(_E):
            idx1 = jnp.minimum(
                idx1, jnp.where(ls[e] == m1, jnp.float32(e), 16.0))
        neg = jnp.float32(-3.0e38)
        m2 = jnp.full((16,), neg, jnp.float32)
        for e in range(_E):
            m2 = jnp.maximum(m2, jnp.where(idx1 == e, neg, ls[e]))
        idx2 = jnp.full((16,), 16.0, jnp.float32)
        for e in range(_E):
            cand = jnp.where(idx1 == e, neg, ls[e])
            idx2 = jnp.minimum(
                idx2, jnp.where(cand == m2, jnp.float32(e), 16.0))
        t = jnp.exp(m2 - m1)
        g1 = 1.0 / (1.0 + t)
        gbuf[0, toks] = g1
        gbuf[1, toks] = t * g1
        ibuf[0, toks] = idx1.astype(jnp.int32)
        ibuf[1, toks] = idx2.astype(jnp.int32)
        ps = [jnp.exp(l - m1) for l in ls]
        s = ps[0]
        for e in range(1, _E):
            s = s + ps[e]
        inv = 1.0 / s
        for e in range(_E):
            plsc.addupdate(accp.at[e], ps[e] * inv)
            plsc.addupdate(accc.at[e], jnp.where(idx1 == e, 1.0, 0.0))
        return carry

    lax.fori_loop(0, _GROUPS, group, 0)
    pltpu.sync_copy(gbuf, gate_hbm.at[:, pl.ds(base, _CHUNK)])
    pltpu.sync_copy(ibuf, idx_hbm.at[:, pl.ds(base, _CHUNK)])
    pltpu.sync_copy(accp, part_hbm.at[0, wid])
    pltpu.sync_copy(accc, part_hbm.at[1, wid])


def _loss_kernel(part_ref, loss_ref):
    part = part_ref[...]
    p = jnp.sum(part[0], axis=(0, 2))
    f = jnp.sum(part[1], axis=(0, 2))
    loss = _E * jnp.sum(p * f, keepdims=True) / (_NT * _NT)
    loss_ref[...] = loss.reshape(1, 1)


def kernel(x, W):
    lt = pl.pallas_call(
        _matmul_kernel,
        grid=(_STEPS,),
        in_specs=[
            pl.BlockSpec((_BT // 2, _D), lambda i: (2 * i, 0)),
            pl.BlockSpec((_BT // 2, _D), lambda i: (2 * i + 1, 0)),
            pl.BlockSpec((_E, _D), lambda i: (0, 0)),
        ],
        out_specs=pl.BlockSpec((_E, _BT), lambda i: (0, i)),
        out_shape=jax.ShapeDtypeStruct((_E, _NT), jnp.float32),
        compiler_params=pltpu.CompilerParams(
            vmem_limit_bytes=100 * 1024 * 1024
        ),
    )(x, x, W)

    mesh = plsc.VectorSubcoreMesh(core_axis_name="c", subcore_axis_name="s")
    gate_t, idx_t, part = pl.kernel(
        _sc_routing_kernel,
        out_type=[
            jax.ShapeDtypeStruct((2, _NT), jnp.float32),
            jax.ShapeDtypeStruct((2, _NT), jnp.int32),
            jax.ShapeDtypeStruct((2, _NW, 16, 16), jnp.float32),
        ],
        mesh=mesh,
        scratch_types=[
            pltpu.VMEM((_E, _CHUNK), jnp.float32),
            pltpu.VMEM((2, _CHUNK), jnp.float32),
            pltpu.VMEM((2, _CHUNK), jnp.int32),
            pltpu.VMEM((_E, 16), jnp.float32),
            pltpu.VMEM((_E, 16), jnp.float32),
        ],
    )(lt)

    loss = pl.pallas_call(
        _loss_kernel,
        out_shape=jax.ShapeDtypeStruct((1, 1), jnp.float32),
    )(part)
    return gate_t.T, idx_t.T, loss[0, 0]


# speedup vs baseline: 1.5145x; 1.5145x over previous
import jax
import jax.numpy as jnp
from jax.experimental import pallas as pl
from jax.experimental.pallas import tpu as pltpu

_NT = 16384
_D = 2048
_E = 16
_BT = 1024
_STEPS = _NT // _BT


def _gating_kernel(xa_ref, xb_ref, w_ref, gate_ref, idx_ref, loss_ref, acc_ref):
    step = pl.program_id(0)
    w = w_ref[...]
    lta = jax.lax.dot_general(
        w, xa_ref[...], (((1,), (1,)), ((), ())), preferred_element_type=jnp.float32
    )
    ltb = jax.lax.dot_general(
        w, xb_ref[...], (((1,), (1,)), ((), ())), preferred_element_type=jnp.float32
    )
    lt = jnp.concatenate([lta, ltb], axis=1)
    iota = jax.lax.broadcasted_iota(jnp.int32, lt.shape, 0)
    m1 = jnp.max(lt, axis=0, keepdims=True)
    is1 = lt == m1
    idx1 = jnp.min(jnp.where(is1, iota, _E), axis=0, keepdims=True)
    masked = jnp.where(iota == idx1, -jnp.inf, lt)
    m2 = jnp.max(masked, axis=0, keepdims=True)
    idx2 = jnp.min(jnp.where(masked == m2, iota, _E), axis=0, keepdims=True)
    t = jnp.exp(m2 - m1)
    denom = 1.0 + t
    gate_ref[...] = jnp.concatenate([1.0 / denom, t / denom], axis=0)
    idx_ref[...] = jnp.concatenate([idx1, idx2], axis=0)
    e = jnp.exp(lt - m1)
    p = e / jnp.sum(e, axis=0, keepdims=True)
    psum = jnp.sum(p, axis=1)
    csum = jnp.sum((iota == idx1).astype(jnp.float32), axis=1)
    part = jnp.stack([psum, csum])

    @pl.when(step == 0)
    def _init():
        acc_ref[...] = part

    @pl.when(step != 0)
    def _accum():
        acc_ref[...] += part

    @pl.when(step == _STEPS - 1)
    def _finalize():
        acc = acc_ref[...]
        loss = _E * jnp.sum(acc[0] * acc[1], keepdims=True) / (_NT * _NT)
        loss_ref[...] = loss.reshape(1, 1)


def kernel(x, W):
    gate_t, idx_t, loss = pl.pallas_call(
        _gating_kernel,
        grid=(_STEPS,),
        in_specs=[
            pl.BlockSpec((_BT // 2, _D), lambda i: (2 * i, 0)),
            pl.BlockSpec((_BT // 2, _D), lambda i: (2 * i + 1, 0)),
            pl.BlockSpec((_E, _D), lambda i: (0, 0)),
        ],
        out_specs=[
            pl.BlockSpec((2, _BT), lambda i: (0, i)),
            pl.BlockSpec((2, _BT), lambda i: (0, i)),
            pl.BlockSpec((1, 1), lambda i: (0, 0)),
        ],
        out_shape=[
            jax.ShapeDtypeStruct((2, _NT), jnp.float32),
            jax.ShapeDtypeStruct((2, _NT), jnp.int32),
            jax.ShapeDtypeStruct((1, 1), jnp.float32),
        ],
        scratch_shapes=[pltpu.VMEM((2, _E), jnp.float32)],
        compiler_params=pltpu.CompilerParams(
            vmem_limit_bytes=100 * 1024 * 1024
        ),
    )(x, x, W)
    return gate_t.T, idx_t.T, loss[0, 0]
